# baseline (speedup 1.0000x reference)
import functools

import jax
import jax.numpy as jnp
from jax import lax
from jax.experimental import pallas as pl
from jax.experimental.pallas import tpu as pltpu
from jax.experimental.pallas import tpu_sc as plsc

_N = 10000
_D = 128
_E = 320000
_EPS = 1e-7
_MIN_NORM = 1e-15

_NC = 2
_NS = 16
_L = 16
_DH = _D // _NC
_K = 128
_NCHT = _E // _K
_CPT = 156
_NEX = _NCHT - _CPT * _NS
_NB = 6
_NI = 78
_NHALF = 13
_RB = 624
_TAIL = _N - _RB * _NS

_R = 2000


def _pre_body(x_ref, o_ref):
    x = x_ref[...]
    col = lax.broadcasted_iota(jnp.int32, x.shape, 1)
    y = jnp.where(col == 0, 0.0, x)
    s = jnp.sum(y * y, axis=1, keepdims=True)
    theta = jnp.maximum(jnp.sqrt(1.0 + s), 1.0 + _EPS)
    y_norm = jnp.maximum(jnp.sqrt(s), _MIN_NORM)
    ach = jnp.log(theta + jnp.sqrt(theta * theta - 1.0))
    t = y * (ach / y_norm)
    o_ref[0] = t[:, :_DH]
    o_ref[1] = t[:, _DH:]


_pre_tc = pl.pallas_call(
    _pre_body,
    grid=(_N // _R,),
    in_specs=[pl.BlockSpec((_R, _D), lambda i: (i, 0))],
    out_specs=pl.BlockSpec((_NC, _R, _DH), lambda i: (0, i, 0)),
    out_shape=jax.ShapeDtypeStruct((_NC, _N, _DH), jnp.float32),
)


def _post_body(a_ref, b_ref, c_ref, o_ref):
    g = a_ref[...] + b_ref[...] + c_ref[...]
    t = jnp.concatenate([g[0], g[1]], axis=1)
    s = jnp.sum(t * t, axis=1, keepdims=True)
    xn = jnp.maximum(jnp.sqrt(s), _MIN_NORM)
    sh = 0.5 * (jnp.exp(xn) - jnp.exp(-xn))
    rest = t * (sh / xn)
    s2 = jnp.sum(rest * rest, axis=1, keepdims=True)
    first = jnp.sqrt(jnp.maximum(1.0 + s2, _EPS))
    col = lax.broadcasted_iota(jnp.int32, t.shape, 1)
    o_ref[...] = jnp.where(col == 0, first, rest)


_post_tc = pl.pallas_call(
    _post_body,
    grid=(_N // _R,),
    in_specs=[pl.BlockSpec((_NC, _R, _DH), lambda i: (0, i, 0))] * 3,
    out_specs=pl.BlockSpec((_R, _D), lambda i: (i, 0)),
    out_shape=jax.ShapeDtypeStruct((_N, _D), jnp.float32),
)


_mesh = plsc.VectorSubcoreMesh(core_axis_name="c", subcore_axis_name="s")


def _bcast_lane(wv, j):
    return lax.gather(
        wv, jnp.full((_L, 1), j, jnp.int32),
        lax.GatherDimensionNumbers(
            offset_dims=(), collapsed_slice_dims=(0,), start_index_map=(0,)),
        slice_sizes=(1,),
        mode=lax.GatherScatterMode.PROMISE_IN_BOUNDS)


@functools.partial(
    pl.kernel,
    out_type=jax.ShapeDtypeStruct((_NC * _N, _DH), jnp.float32),
    mesh=_mesh,
    compiler_params=pltpu.CompilerParams(use_tc_tiling_on_sc=False),
    scratch_types=[
        pltpu.VMEM((_NB, _K, _DH), jnp.float32),
        pltpu.VMEM((_NI, _K), jnp.int32),
        pltpu.VMEM((_NI, _K), jnp.int32),
        pltpu.VMEM((_NI, _K), jnp.float32),
        pltpu.VMEM_SHARED((_N, _DH), jnp.float32),
        pltpu.SemaphoreType.DMA((_NB,)),
        pltpu.SemaphoreType.DMA((_NB,)),
    ],
)
def _spmm_sc(x_hbm, src_hbm, dst_hbm, w_hbm, out_hbm,
             rows_v, srcw, dstw, wring, acc_sh, gsem, ssem):
    cid = lax.axis_index("c")
    sid = lax.axis_index("s")
    tb = sid * _CPT
    src_off = cid * _N

    zero = jnp.zeros((_L,), jnp.float32)

    def _zrow(i, carry):
        for c in range(_DH // _L):
            rows_v[0, i, pl.ds(c * _L, _L)] = zero
        return carry

    lax.fori_loop(0, _K, _zrow, 0)
    for p in range(4):
        pltpu.sync_copy(rows_v.at[0],
                        acc_sh.at[pl.ds(sid * _RB + p * _K, _K)])
    pltpu.sync_copy(rows_v.at[0, pl.ds(0, _RB - 4 * _K)],
                    acc_sh.at[pl.ds(sid * _RB + 4 * _K, _RB - 4 * _K)])

    @pl.when(sid == _NS - 1)
    def _zero_tail():
        pltpu.sync_copy(rows_v.at[0, pl.ds(0, _TAIL)],
                        acc_sh.at[pl.ds(_RB * _NS, _TAIL)])

    plsc.subcore_barrier()

    def _shift_src(islot):
        for g in range(_K // _L):
            sl = pl.ds(g * _L, _L)
            srcw[islot, sl] = srcw[islot, sl] + src_off

    def _scale(islot, r):
        @plsc.parallel_loop(0, _K // _L, unroll=2)
        def _grp(g):
            wv = wring[islot, pl.ds(g * _L, _L)]
            for j in range(_L):
                wb = _bcast_lane(wv, j)
                e = g * _L + j
                for c in range(_DH // _L):
                    sl = pl.ds(c * _L, _L)
                    rows_v[r, e, sl] = rows_v[r, e, sl] * wb


    def _shift_all(k, carry):
        _shift_src(k)
        return carry

    def _half(hh, carry):
        ibase = 6 * hh
        gd = []
        for b in range(_NB):
            gd.append(pltpu.async_copy(
                x_hbm.at[srcw.at[ibase + b]], rows_v.at[b], gsem.at[b]))
        sd = []
        for b in range(_NB):
            gd[b].wait()
            _scale(ibase + b, b)
            sd.append(pltpu.async_copy(
                rows_v.at[b], acc_sh.at[dstw.at[ibase + b]], ssem.at[b],
                add=True))
        for b in range(_NB):
            sd[b].wait()
        return carry

    def _iter(i, carry):
        row0 = tb + _NI * i
        pltpu.sync_copy(src_hbm.at[pl.ds(row0, _NI)], srcw)
        pltpu.sync_copy(dst_hbm.at[pl.ds(row0, _NI)], dstw)
        pltpu.sync_copy(w_hbm.at[pl.ds(row0, _NI)], wring)
        lax.fori_loop(0, _NI, _shift_all, 0)
        lax.fori_loop(0, _NHALF, _half, 0)
        return carry

    lax.fori_loop(0, 2, _iter, 0)

    @pl.when(sid < _NEX)
    def _extra():
        gr = _CPT * _NS + sid
        pltpu.sync_copy(src_hbm.at[pl.ds(gr, 1)], srcw.at[pl.ds(0, 1)])
        pltpu.sync_copy(dst_hbm.at[pl.ds(gr, 1)], dstw.at[pl.ds(0, 1)])
        pltpu.sync_copy(w_hbm.at[pl.ds(gr, 1)], wring.at[pl.ds(0, 1)])

        _shift_src(0)
        pltpu.async_copy(x_hbm.at[srcw.at[0]], rows_v.at[0],
                         gsem.at[0]).wait()
        _scale(0, 0)
        pltpu.sync_copy(rows_v.at[0], acc_sh.at[dstw.at[0]], add=True)

    plsc.subcore_barrier()

    row_off = cid * _N + sid * _RB
    for p in range(4):
        pltpu.sync_copy(acc_sh.at[pl.ds(sid * _RB + p * _K, _K)],
                        rows_v.at[p])
        pltpu.sync_copy(rows_v.at[p], out_hbm.at[pl.ds(row_off + p * _K, _K)])
    pltpu.sync_copy(acc_sh.at[pl.ds(sid * _RB + 4 * _K, _RB - 4 * _K)],
                    rows_v.at[4, pl.ds(0, _RB - 4 * _K)])
    pltpu.sync_copy(rows_v.at[4, pl.ds(0, _RB - 4 * _K)],
                    out_hbm.at[pl.ds(row_off + 4 * _K, _RB - 4 * _K)])

    @pl.when(sid == _NS - 1)
    def _write_tail():
        pltpu.sync_copy(acc_sh.at[pl.ds(_RB * _NS, _TAIL)],
                        rows_v.at[5, pl.ds(0, _TAIL)])
        pltpu.sync_copy(rows_v.at[5, pl.ds(0, _TAIL)],
                        out_hbm.at[pl.ds(cid * _N + _RB * _NS, _TAIL)])


def kernel(x, edge_index, edge_weight):
    src2d = edge_index[1].reshape(_NCHT, _K)
    dst2d = edge_index[0].reshape(_NCHT, _K)
    ws = edge_weight.reshape(_NCHT, _K)
    xt = _pre_tc(x).reshape(_NC * _N, _DH)
    o1 = _spmm_sc(xt, src2d, dst2d, ws)
    o2 = _spmm_sc(o1, src2d, dst2d, ws)
    o3 = _spmm_sc(o2, src2d, dst2d, ws)
    return _post_tc(
        o1.reshape(_NC, _N, _DH),
        o2.reshape(_NC, _N, _DH),
        o3.reshape(_NC, _N, _DH),
    )
